# baseline (speedup 1.0000x reference)
import functools

import jax
import jax.numpy as jnp
from jax import lax
from jax.experimental import pallas as pl
from jax.experimental.pallas import tpu as pltpu
from jax.experimental.pallas import tpu_sc as plsc

B = 16384
D = 64
NW = 32
BW = B // NW
NCHUNK = 4
CW = BW // NCHUNK
L = 16
RING = 4
NTAB = 3


def _sc_partials(h, t, r, ent2, rel2):
    mesh = plsc.VectorSubcoreMesh(core_axis_name="c", subcore_axis_name="s")

    @functools.partial(
        pl.kernel,
        mesh=mesh,
        out_type=jax.ShapeDtypeStruct((NTAB, NW, BW // 8, 128), jnp.float32),
        scratch_types=[
            [pltpu.VMEM((NCHUNK, CW), jnp.int32) for _ in range(NTAB)],
            [pltpu.VMEM((NCHUNK, CW), jnp.int32) for _ in range(NTAB)],
            [pltpu.VMEM((BW + L,), jnp.int32) for _ in range(NTAB)],
            [pltpu.VMEM((BW // 8, 128), jnp.float32) for _ in range(NTAB)],
            pltpu.VMEM((RING, CW, 128), jnp.float32),
            [pltpu.SemaphoreType.DMA for _ in range(RING)],
        ],
    )
    def sc_kernel(h_hbm, t_hbm, r_hbm, ent_hbm, rel_hbm, out_hbm,
                  raws, packs, offs, parts, ring, sems):
        wid = lax.axis_index("s") * 2 + lax.axis_index("c")
        base = wid * BW

        for k, src in enumerate((h_hbm, t_hbm, r_hbm)):
            for c in range(NCHUNK):
                pltpu.sync_copy(src.at[pl.ds(base + c * CW, CW)], raws[k].at[c])
            for c in range(NCHUNK):
                for j in range(CW // L):
                    v = raws[k][c, pl.ds(j * L, L)]
                    packs[k][c, pl.ds(j * L, L)] = lax.shift_right_logical(v, 1)
                    offs[k][pl.ds(c * CW + j * L, L)] = (v & 1) * D

        tables = (ent_hbm, ent_hbm, rel_hbm)
        chunks = [(k, c) for k in range(NTAB) for c in range(NCHUNK)]
        handles = [None] * len(chunks)

        def fire(j):
            k, c = chunks[j]
            slot = j % RING
            handles[j] = pltpu.async_copy(
                tables[k].at[packs[k].at[c]], ring.at[slot], sems[slot])

        for j in range(RING):
            fire(j)

        for j, (k, c) in enumerate(chunks):
            handles[j].wait()
            slot = j % RING
            part, off = parts[k], offs[k]

            def body(i, carry, _slot=slot, _part=part, _off=off, _c=c):
                rows = ring.at[_slot]
                g = _c * CW + i
                o = _off[pl.ds(g, L)][0]
                v0 = rows[i, pl.ds(o, L)]
                v1 = rows[i, pl.ds(o + L, L)]
                v2 = rows[i, pl.ds(o + 2 * L, L)]
                v3 = rows[i, pl.ds(o + 3 * L, L)]
                _part[g // 8, pl.ds((g % 8) * L, L)] = (
                    v0 * v0 + v1 * v1 + v2 * v2 + v3 * v3)
                return carry

            lax.fori_loop(0, CW, body, 0, unroll=8)
            if j + RING < len(chunks):
                fire(j + RING)

        for k in range(NTAB):
            pltpu.sync_copy(parts[k], out_hbm.at[k, wid])

    return sc_kernel(h, t, r, ent2, rel2)


def _tc_combine(p):
    def tc_kernel(p_ref, o_ref):
        col = lax.broadcasted_iota(jnp.int32, (128, 8), 0) // 16
        grp = lax.broadcasted_iota(jnp.int32, (128, 8), 1)
        m = (col == grp).astype(jnp.float32)
        sh = jnp.dot(p_ref[0], m, preferred_element_type=jnp.float32)
        st = jnp.dot(p_ref[1], m, preferred_element_type=jnp.float32)
        sr = jnp.dot(p_ref[2], m, preferred_element_type=jnp.float32)
        d = jnp.sqrt(sh) - jnp.sqrt(st) + jnp.sqrt(sr)
        o_ref[...] = jnp.sqrt(jnp.sum(d * d)).reshape(1, 1)

    return pl.pallas_call(
        tc_kernel,
        out_shape=jax.ShapeDtypeStruct((1, 1), jnp.float32),
    )(p)


def kernel(h, r, t, emb_entity, emb_relation, norm_p):
    ent2 = emb_entity.reshape(-1, 128)
    rel2 = emb_relation.reshape(-1, 128)
    parts = _sc_partials(h, t, r, ent2, rel2)
    parts = parts.reshape(3, B * L // 128, 128)
    out = _tc_combine(parts)[0, 0]
    pf = jnp.asarray(norm_p, jnp.float32)
    return out * (pf / pf)


# speedup vs baseline: 1.0430x; 1.0430x over previous
import functools

import jax
import jax.numpy as jnp
from jax import lax
from jax.experimental import pallas as pl
from jax.experimental.pallas import tpu as pltpu
from jax.experimental.pallas import tpu_sc as plsc

B = 16384
D = 64
NE = 1000000
NW = 32
BW = B // NW
NCHUNK = 4
CW = BW // NCHUNK
L = 16
RING = 4
NTAB = 3
W = 1024
GRID = -(-NE // W)


def _norms_body(x_ref, o_ref):
    x = x_ref[...]
    for w in range(o_ref.shape[0]):
        sq = x[:, w * 128:(w + 1) * 128]
        o_ref[pl.ds(w, 1), :] = jnp.sum(sq * sq, axis=0, keepdims=True)


def _tc_entity_norms(tt):
    return pl.pallas_call(
        _norms_body,
        grid=(GRID,),
        in_specs=[pl.BlockSpec((D, W), lambda c: (0, c))],
        out_specs=pl.BlockSpec((W // 128, 128), lambda c: (c, 0)),
        out_shape=jax.ShapeDtypeStruct((GRID * W // 128, 128), jnp.float32),
    )(tt)


def _tc_relation_norms(rt_pad):
    return pl.pallas_call(
        _norms_body,
        out_shape=jax.ShapeDtypeStruct((8, 128), jnp.float32),
    )(rt_pad)


def _sc_gather(h, t, r, s2, sr):
    mesh = plsc.VectorSubcoreMesh(core_axis_name="c", subcore_axis_name="s")

    @functools.partial(
        pl.kernel,
        mesh=mesh,
        out_type=jax.ShapeDtypeStruct((NTAB, B // 128, 128), jnp.float32),
        compiler_params=pltpu.CompilerParams(needs_layout_passes=False),
        scratch_types=[
            [pltpu.VMEM((NCHUNK, CW), jnp.int32) for _ in range(NTAB)],
            [pltpu.VMEM((NCHUNK, CW), jnp.int32) for _ in range(NTAB)],
            [pltpu.VMEM((BW // 128, 128), jnp.float32) for _ in range(NTAB)],
            pltpu.VMEM((RING, CW, 128), jnp.float32),
            [pltpu.SemaphoreType.DMA for _ in range(RING)],
        ],
    )
    def sc_kernel(h_hbm, t_hbm, r_hbm, s2_hbm, sr_hbm, out_hbm,
                  raws, rowidx, vals, ring, sems):
        wid = lax.axis_index("s") * 2 + lax.axis_index("c")
        base = wid * BW

        for k, src in enumerate((h_hbm, t_hbm, r_hbm)):
            for c in range(NCHUNK):
                pltpu.sync_copy(src.at[pl.ds(base + c * CW, CW)], raws[k].at[c])
            for c in range(NCHUNK):
                for j in range(CW // L):
                    v = raws[k][c, pl.ds(j * L, L)]
                    rowidx[k][c, pl.ds(j * L, L)] = lax.shift_right_logical(v, 7)

        tables = (s2_hbm, s2_hbm, sr_hbm)
        chunks = [(k, c) for k in range(NTAB) for c in range(NCHUNK)]
        handles = [None] * len(chunks)

        def fire(j):
            k, c = chunks[j]
            slot = j % RING
            handles[j] = pltpu.async_copy(
                tables[k].at[rowidx[k].at[c]], ring.at[slot], sems[slot])

        for j in range(RING):
            fire(j)

        lane = lax.broadcasted_iota(jnp.int32, (L,), 0)
        for j, (k, c) in enumerate(chunks):
            handles[j].wait()
            slot = j % RING
            for g in range(CW // L):
                v = raws[k][c, pl.ds(g * L, L)]
                row_local = lane + (g * L)
                col = v & 127
                val = plsc.load_gather(ring.at[slot], [row_local, col])
                pos = c * CW + g * L
                vals[k][pos // 128, pl.ds(pos % 128, L)] = val
            if j + RING < len(chunks):
                fire(j + RING)

        for k in range(NTAB):
            pltpu.sync_copy(vals[k], out_hbm.at[k, pl.ds(wid * (BW // 128),
                                                         BW // 128)])

    return sc_kernel(h, t, r, s2, sr)


def _tc_combine(p):
    def body(p_ref, o_ref):
        d = (jnp.sqrt(p_ref[0]) - jnp.sqrt(p_ref[1]) + jnp.sqrt(p_ref[2]))
        o_ref[...] = jnp.sqrt(jnp.sum(d * d)).reshape(1, 1)

    return pl.pallas_call(
        body,
        out_shape=jax.ShapeDtypeStruct((1, 1), jnp.float32),
    )(p)


def kernel(h, r, t, emb_entity, emb_relation, norm_p):
    s2 = _tc_entity_norms(emb_entity.T)
    sr = _tc_relation_norms(jnp.pad(emb_relation.T, ((0, 0), (0, 24))))
    p = _sc_gather(h, t, r, s2, sr)
    out = _tc_combine(p)[0, 0]
    pf = jnp.asarray(norm_p, jnp.float32)
    return out * (pf / pf)


# speedup vs baseline: 4.6091x; 4.4191x over previous
import functools

import jax
import jax.numpy as jnp
from jax import lax
from jax.experimental import pallas as pl
from jax.experimental.pallas import tpu as pltpu
from jax.experimental.pallas import tpu_sc as plsc

B = 16384
D = 64
NE = 1000000
NW = 32
BW = B // NW
NCHUNK = 4
CW = BW // NCHUNK
L = 16
W = 8192
GRID = -(-NE // W)


def _norms_body(x_ref, o_ref):
    x = x_ref[...]
    for w in range(o_ref.shape[0]):
        sq = x[:, w * 128:(w + 1) * 128]
        o_ref[pl.ds(w, 1), :] = jnp.sum(sq * sq, axis=0, keepdims=True)


def _tc_entity_norms(tt):
    return pl.pallas_call(
        _norms_body,
        grid=(GRID,),
        in_specs=[pl.BlockSpec((D, W), lambda c: (0, c))],
        out_specs=pl.BlockSpec((W // 128, 128), lambda c: (c, 0)),
        out_shape=jax.ShapeDtypeStruct((GRID * W // 128, 128), jnp.float32),
    )(tt)


def _tc_relation_norms(rt_pad):
    return pl.pallas_call(
        _norms_body,
        out_shape=jax.ShapeDtypeStruct((8, 128), jnp.float32),
    )(rt_pad)


def _sc_gather(h, t, r, s1d, sr1d):
    mesh = plsc.VectorSubcoreMesh(core_axis_name="c", subcore_axis_name="s")

    @functools.partial(
        pl.kernel,
        mesh=mesh,
        out_type=jax.ShapeDtypeStruct((3 * B,), jnp.float32),
        compiler_params=pltpu.CompilerParams(needs_layout_passes=False),
        scratch_types=[
            [pltpu.VMEM((BW,), jnp.int32) for _ in range(3)],
            [pltpu.VMEM((BW,), jnp.float32) for _ in range(3)],
            pltpu.VMEM((1024,), jnp.float32),
            pltpu.SemaphoreType.DMA,
        ],
    )
    def sc_kernel(h_hbm, t_hbm, r_hbm, s_hbm, sr_hbm, out_hbm,
                  raws, vals, srv, sem):
        wid = lax.axis_index("s") * 2 + lax.axis_index("c")
        base = wid * BW

        for k, src in enumerate((h_hbm, t_hbm, r_hbm)):
            pltpu.sync_copy(src.at[pl.ds(base, BW)], raws[k])
        pltpu.sync_copy(sr_hbm, srv)

        copies = []
        for k in range(2):
            for c in range(NCHUNK):
                sl = pl.ds(c * CW, CW)
                copies.append(pltpu.async_copy(
                    s_hbm.at[raws[k].at[sl]], vals[k].at[sl], sem))

        for g in range(BW // L):
            v = raws[2][pl.ds(g * L, L)]
            vals[2][pl.ds(g * L, L)] = plsc.load_gather(srv, [v])

        for cp in copies:
            cp.wait()
        for k in range(3):
            pltpu.sync_copy(vals[k], out_hbm.at[pl.ds(k * B + base, BW)])

    return sc_kernel(h, t, r, s1d, sr1d)


def _tc_combine(p):
    def body(p_ref, o_ref):
        n = p_ref.shape[0] // 3
        d = (jnp.sqrt(p_ref[0:n, :]) - jnp.sqrt(p_ref[n:2 * n, :])
             + jnp.sqrt(p_ref[2 * n:3 * n, :]))
        o_ref[...] = jnp.sqrt(jnp.sum(d * d)).reshape(1, 1)

    return pl.pallas_call(
        body,
        out_shape=jax.ShapeDtypeStruct((1, 1), jnp.float32),
    )(p)


def kernel(h, r, t, emb_entity, emb_relation, norm_p):
    s2 = _tc_entity_norms(emb_entity.T)
    sr = _tc_relation_norms(jnp.pad(emb_relation.T, ((0, 0), (0, 24))))
    p = _sc_gather(h, t, r, s2.reshape(-1), sr.reshape(-1))
    out = _tc_combine(p.reshape(3 * B // 128, 128))[0, 0]
    pf = jnp.asarray(norm_p, jnp.float32)
    return out * (pf / pf)


# speedup vs baseline: 5.8885x; 1.2776x over previous
import functools

import jax
import jax.numpy as jnp
from jax import lax
from jax.experimental import pallas as pl
from jax.experimental.pallas import tpu as pltpu
from jax.experimental.pallas import tpu_sc as plsc

B = 16384
D = 64
NE = 1000000
NW = 32
BW = B // NW
NCHUNK = 4
CW = BW // NCHUNK
L = 16
W = 16384
GRID = -(-NE // W)


def _norms_body(x_ref, o_ref):
    x = x_ref[...]
    for w in range(o_ref.shape[0]):
        sq = x[:, w * 128:(w + 1) * 128]
        o_ref[pl.ds(w, 1), :] = jnp.sum(sq * sq, axis=0, keepdims=True)


def _tc_entity_norms(tt):
    return pl.pallas_call(
        _norms_body,
        grid=(GRID,),
        in_specs=[pl.BlockSpec((D, W), lambda c: (0, c))],
        out_specs=pl.BlockSpec((W // 128, 128), lambda c: (c, 0)),
        out_shape=jax.ShapeDtypeStruct((GRID * W // 128, 128), jnp.float32),
    )(tt)


def _tc_relation_norms(rt_pad):
    return pl.pallas_call(
        _norms_body,
        out_shape=jax.ShapeDtypeStruct((8, 128), jnp.float32),
    )(rt_pad)


def _sc_gather(h, t, r, s1d, sr1d):
    mesh = plsc.VectorSubcoreMesh(core_axis_name="c", subcore_axis_name="s")

    @functools.partial(
        pl.kernel,
        mesh=mesh,
        out_type=jax.ShapeDtypeStruct((3 * B,), jnp.float32),
        compiler_params=pltpu.CompilerParams(needs_layout_passes=False),
        scratch_types=[
            [pltpu.VMEM((BW,), jnp.int32) for _ in range(3)],
            [pltpu.VMEM((BW,), jnp.float32) for _ in range(3)],
            pltpu.VMEM((1024,), jnp.float32),
            pltpu.SemaphoreType.DMA,
        ],
    )
    def sc_kernel(h_hbm, t_hbm, r_hbm, s_hbm, sr_hbm, out_hbm,
                  raws, vals, srv, sem):
        wid = lax.axis_index("s") * 2 + lax.axis_index("c")
        base = wid * BW

        for k, src in enumerate((h_hbm, t_hbm, r_hbm)):
            pltpu.sync_copy(src.at[pl.ds(base, BW)], raws[k])
        pltpu.sync_copy(sr_hbm, srv)

        copies = []
        for k in range(2):
            for c in range(NCHUNK):
                sl = pl.ds(c * CW, CW)
                copies.append(pltpu.async_copy(
                    s_hbm.at[raws[k].at[sl]], vals[k].at[sl], sem))

        for g in range(BW // L):
            v = raws[2][pl.ds(g * L, L)]
            vals[2][pl.ds(g * L, L)] = plsc.load_gather(srv, [v])

        for cp in copies:
            cp.wait()
        for k in range(3):
            pltpu.sync_copy(vals[k], out_hbm.at[pl.ds(k * B + base, BW)])

    return sc_kernel(h, t, r, s1d, sr1d)


def _tc_combine(p):
    def body(p_ref, o_ref):
        n = p_ref.shape[0] // 3
        d = (jnp.sqrt(p_ref[0:n, :]) - jnp.sqrt(p_ref[n:2 * n, :])
             + jnp.sqrt(p_ref[2 * n:3 * n, :]))
        o_ref[...] = jnp.sqrt(jnp.sum(d * d)).reshape(1, 1)

    return pl.pallas_call(
        body,
        out_shape=jax.ShapeDtypeStruct((1, 1), jnp.float32),
    )(p)


def kernel(h, r, t, emb_entity, emb_relation, norm_p):
    s2 = _tc_entity_norms(emb_entity.T)
    sr = _tc_relation_norms(jnp.pad(emb_relation.T, ((0, 0), (0, 24))))
    p = _sc_gather(h, t, r, s2.reshape(-1), sr.reshape(-1))
    out = _tc_combine(p.reshape(3 * B // 128, 128))[0, 0]
    pf = jnp.asarray(norm_p, jnp.float32)
    return out * (pf / pf)


# speedup vs baseline: 6.4201x; 1.0903x over previous
import functools

import jax
import jax.numpy as jnp
from jax import lax
from jax.experimental import pallas as pl
from jax.experimental.pallas import tpu as pltpu
from jax.experimental.pallas import tpu_sc as plsc

B = 16384
D = 64
NE = 1000000
NW = 32
BW = B // NW
NCHUNK = 4
CW = BW // NCHUNK
L = 16
W = 32768
GRID = -(-NE // W)


def _norms_body(x_ref, o_ref):
    x = x_ref[...]
    for w in range(o_ref.shape[0]):
        sq = x[:, w * 128:(w + 1) * 128]
        o_ref[pl.ds(w, 1), :] = jnp.sum(sq * sq, axis=0, keepdims=True)


def _tc_entity_norms(tt):
    return pl.pallas_call(
        _norms_body,
        grid=(GRID,),
        in_specs=[pl.BlockSpec((D, W), lambda c: (0, c))],
        out_specs=pl.BlockSpec((W // 128, 128), lambda c: (c, 0)),
        out_shape=jax.ShapeDtypeStruct((GRID * W // 128, 128), jnp.float32),
    )(tt)


def _tc_relation_norms(rt_pad):
    return pl.pallas_call(
        _norms_body,
        out_shape=jax.ShapeDtypeStruct((8, 128), jnp.float32),
    )(rt_pad)


def _sc_gather(h, t, r, s1d, sr1d):
    mesh = plsc.VectorSubcoreMesh(core_axis_name="c", subcore_axis_name="s")

    @functools.partial(
        pl.kernel,
        mesh=mesh,
        out_type=jax.ShapeDtypeStruct((3 * B,), jnp.float32),
        compiler_params=pltpu.CompilerParams(needs_layout_passes=False),
        scratch_types=[
            [pltpu.VMEM((BW,), jnp.int32) for _ in range(3)],
            [pltpu.VMEM((BW,), jnp.float32) for _ in range(3)],
            pltpu.VMEM((1024,), jnp.float32),
            pltpu.SemaphoreType.DMA,
        ],
    )
    def sc_kernel(h_hbm, t_hbm, r_hbm, s_hbm, sr_hbm, out_hbm,
                  raws, vals, srv, sem):
        wid = lax.axis_index("s") * 2 + lax.axis_index("c")
        base = wid * BW

        for k, src in enumerate((h_hbm, t_hbm, r_hbm)):
            pltpu.sync_copy(src.at[pl.ds(base, BW)], raws[k])
        pltpu.sync_copy(sr_hbm, srv)

        copies = []
        for k in range(2):
            for c in range(NCHUNK):
                sl = pl.ds(c * CW, CW)
                copies.append(pltpu.async_copy(
                    s_hbm.at[raws[k].at[sl]], vals[k].at[sl], sem))

        for g in range(BW // L):
            v = raws[2][pl.ds(g * L, L)]
            vals[2][pl.ds(g * L, L)] = plsc.load_gather(srv, [v])

        for cp in copies:
            cp.wait()
        for k in range(3):
            pltpu.sync_copy(vals[k], out_hbm.at[pl.ds(k * B + base, BW)])

    return sc_kernel(h, t, r, s1d, sr1d)


def _tc_combine(p):
    def body(p_ref, o_ref):
        n = p_ref.shape[0] // 3
        d = (jnp.sqrt(p_ref[0:n, :]) - jnp.sqrt(p_ref[n:2 * n, :])
             + jnp.sqrt(p_ref[2 * n:3 * n, :]))
        o_ref[...] = jnp.sqrt(jnp.sum(d * d)).reshape(1, 1)

    return pl.pallas_call(
        body,
        out_shape=jax.ShapeDtypeStruct((1, 1), jnp.float32),
    )(p)


def kernel(h, r, t, emb_entity, emb_relation, norm_p):
    s2 = _tc_entity_norms(emb_entity.T)
    sr = _tc_relation_norms(jnp.pad(emb_relation.T, ((0, 0), (0, 24))))
    p = _sc_gather(h, t, r, s2.reshape(-1), sr.reshape(-1))
    out = _tc_combine(p.reshape(3 * B // 128, 128))[0, 0]
    pf = jnp.asarray(norm_p, jnp.float32)
    return out * (pf / pf)
